# baseline (speedup 1.0000x reference)
import functools

import jax
import jax.numpy as jnp
from jax import lax
from jax.experimental import pallas as pl
from jax.experimental.pallas import tpu as pltpu
from jax.experimental.pallas import tpu_sc as plsc

N = 10000
E = 320000
D = 128
NC = 2
NS = 16
NW = NC * NS
K = 80
CHUNKS = 126
CPP = CHUNKS // 2
EPW = CHUNKS * K
E_PAD = EPW * NW
N_PAD = 10240
ZR = N_PAD // NS

_mesh = plsc.VectorSubcoreMesh(
    core_axis_name="c", subcore_axis_name="s", num_cores=NC, num_subcores=NS
)


@functools.partial(
    pl.kernel,
    out_type=jax.ShapeDtypeStruct((NC, N_PAD, D), jnp.float32),
    mesh=_mesh,
    scratch_types=[
        pltpu.VMEM((CPP, K), jnp.int32),
        pltpu.VMEM((CPP, K), jnp.int32),
        [pltpu.VMEM((K, D), jnp.float32) for _ in range(3)],
        pltpu.VMEM_SHARED((N_PAD, D), jnp.float32),
        [pltpu.SemaphoreType.DMA for _ in range(3)],
        [pltpu.SemaphoreType.DMA for _ in range(3)],
    ],
)
def _sc_agg(h_hbm, src_hbm, dst_hbm, zeros_hbm, out_hbm,
            src_v, dst_v, bufs, acc, gsems, ssems):
    c = lax.axis_index("c")
    s = lax.axis_index("s")
    w = s * NC + c
    pltpu.sync_copy(zeros_hbm, acc.at[pl.ds(s * ZR, ZR)])

    def _fire_g(j, b):
        pltpu.async_copy(h_hbm.at[src_v.at[j]], bufs[b], gsems[b])

    def _step(j, b, wait_prev=True, fire_next=True):
        bn = (b + 1) % 3
        pltpu.make_async_copy(h_hbm.at[src_v.at[j]], bufs[b], gsems[b]).wait()
        pltpu.async_copy(bufs[b], acc.at[dst_v.at[j]], ssems[b], add=True)
        if wait_prev:
            pltpu.make_async_copy(
                bufs[bn], acc.at[dst_v.at[j - 2]], ssems[bn]).wait()
        if fire_next:
            _fire_g(j + 1, bn)

    def _wait_s(j, b):
        pltpu.make_async_copy(bufs[b], acc.at[dst_v.at[j]], ssems[b]).wait()

    plsc.subcore_barrier()

    for phase in range(CHUNKS // CPP):
        pltpu.sync_copy(src_hbm.at[w, phase], src_v)
        pltpu.sync_copy(dst_hbm.at[w, phase], dst_v)
        _fire_g(0, 0)
        _step(0, 0, wait_prev=False)
        _step(1, 1, wait_prev=False)
        _step(2, 2)

        @pl.loop(3, CPP - 3, step=3)
        def _chunk(j0):
            for l in range(3):
                _step(j0 + l, l)

        _step(CPP - 3, (CPP - 3) % 3)
        _step(CPP - 2, (CPP - 2) % 3)
        _step(CPP - 1, (CPP - 1) % 3, fire_next=False)
        _wait_s(CPP - 2, (CPP - 2) % 3)
        _wait_s(CPP - 1, (CPP - 1) % 3)

    plsc.subcore_barrier()
    pltpu.sync_copy(acc.at[pl.ds(s * ZR, ZR)],
                    out_hbm.at[c, pl.ds(s * ZR, ZR)])


BN = 2000


def _lin_body(x_ref, p_ref, w_ref, b_ref, o_ref, *, relu):
    h = x_ref[...] + p_ref[0] + p_ref[1]
    y = lax.dot_general(h, w_ref[...], (((1,), (1,)), ((), ())),
                        preferred_element_type=jnp.float32)
    y = y + b_ref[...]
    if relu:
        y = jnp.maximum(y, 0.0)
    o_ref[...] = y


def _linear(x, p, w, b, relu):
    return pl.pallas_call(
        functools.partial(_lin_body, relu=relu),
        grid=(N // BN,),
        in_specs=[
            pl.BlockSpec((BN, D), lambda i: (i, 0)),
            pl.BlockSpec((NC, BN, D), lambda i: (0, i, 0)),
            pl.BlockSpec((D, D), lambda i: (0, 0)),
            pl.BlockSpec((1, D), lambda i: (0, 0)),
        ],
        out_specs=pl.BlockSpec((BN, D), lambda i: (i, 0)),
        out_shape=jax.ShapeDtypeStruct((N, D), jnp.float32),
    )(x, p, w, b)


def kernel(x, adj_t, W1, b1, W2, b2):
    src = adj_t[0].astype(jnp.int32)
    dst = adj_t[1].astype(jnp.int32)
    pad = E_PAD - E
    pad_i = jnp.arange(pad, dtype=jnp.int32)
    src = jnp.concatenate([src, pad_i % N]).reshape(NW, 2, CPP, K)
    dst = jnp.concatenate([dst, N + pad_i % (N_PAD - N)]).reshape(NW, 2, CPP, K)
    zeros = jnp.zeros((ZR, D), jnp.float32)
    b1r = b1.reshape(1, D)
    b2r = b2.reshape(1, D)

    p1 = _sc_agg(x, src, dst, zeros)
    h = _linear(x, p1, W1, b1r, relu=True)
    p2 = _sc_agg(h, src, dst, zeros)
    return _linear(h, p2, W2, b2r, relu=False)


# speedup vs baseline: 1.3448x; 1.3448x over previous
import functools

import jax
import jax.numpy as jnp
from jax import lax
from jax.experimental import pallas as pl
from jax.experimental.pallas import tpu as pltpu
from jax.experimental.pallas import tpu_sc as plsc

N = 10000
E = 320000
D = 128
NC = 2
NS = 16
NW = NC * NS
K = 112
CHUNKS = 90
NPH = 5
CPP = CHUNKS // NPH
EPW = CHUNKS * K
E_PAD = EPW * NW
N_PAD = 10112
ZR = N_PAD // NS

_mesh = plsc.VectorSubcoreMesh(
    core_axis_name="c", subcore_axis_name="s", num_cores=NC, num_subcores=NS
)


@functools.partial(
    pl.kernel,
    out_type=jax.ShapeDtypeStruct((NC, N_PAD, D), jnp.float32),
    mesh=_mesh,
    scratch_types=[
        pltpu.VMEM((CPP, K), jnp.int32),
        pltpu.VMEM((CPP, K), jnp.int32),
        [pltpu.VMEM((K, D), jnp.float32) for _ in range(3)],
        pltpu.VMEM_SHARED((N_PAD, D), jnp.float32),
        [pltpu.SemaphoreType.DMA for _ in range(3)],
    ],
)
def _sc_agg(h_hbm, src_hbm, dst_hbm, zeros_hbm, out_hbm,
            src_v, dst_v, bufs, acc, gsems):
    c = lax.axis_index("c")
    s = lax.axis_index("s")
    w = s * NC + c
    pltpu.sync_copy(zeros_hbm, acc.at[pl.ds(s * ZR, ZR)])

    def _fire_g(j, b):
        pltpu.async_copy(h_hbm.at[src_v.at[j]], bufs[b], gsems[b])

    def _step(j, b, fire_next=True):
        if fire_next:
            _fire_g(j + 2, (b + 2) % 3)
        pltpu.make_async_copy(h_hbm.at[src_v.at[j]], bufs[b], gsems[b]).wait()
        pltpu.sync_copy(bufs[b], acc.at[dst_v.at[j]], add=True)

    plsc.subcore_barrier()

    for phase in range(NPH):
        pltpu.sync_copy(src_hbm.at[w, phase], src_v)
        pltpu.sync_copy(dst_hbm.at[w, phase], dst_v)
        _fire_g(0, 0)
        _fire_g(1, 1)

        @pl.loop(0, CPP - 3, step=3)
        def _chunk(j0):
            for l in range(3):
                _step(j0 + l, l)

        _step(CPP - 3, (CPP - 3) % 3)
        _step(CPP - 2, (CPP - 2) % 3, fire_next=False)
        _step(CPP - 1, (CPP - 1) % 3, fire_next=False)

    plsc.subcore_barrier()
    pltpu.sync_copy(acc.at[pl.ds(s * ZR, ZR)],
                    out_hbm.at[c, pl.ds(s * ZR, ZR)])


BN = 2000


def _lin_body(x_ref, p_ref, w_ref, b_ref, o_ref, *, relu):
    h = x_ref[...] + p_ref[0] + p_ref[1]
    y = lax.dot_general(h, w_ref[...], (((1,), (1,)), ((), ())),
                        preferred_element_type=jnp.float32)
    y = y + b_ref[...]
    if relu:
        y = jnp.maximum(y, 0.0)
    o_ref[...] = y


def _linear(x, p, w, b, relu):
    return pl.pallas_call(
        functools.partial(_lin_body, relu=relu),
        grid=(N // BN,),
        in_specs=[
            pl.BlockSpec((BN, D), lambda i: (i, 0)),
            pl.BlockSpec((NC, BN, D), lambda i: (0, i, 0)),
            pl.BlockSpec((D, D), lambda i: (0, 0)),
            pl.BlockSpec((1, D), lambda i: (0, 0)),
        ],
        out_specs=pl.BlockSpec((BN, D), lambda i: (i, 0)),
        out_shape=jax.ShapeDtypeStruct((N, D), jnp.float32),
    )(x, p, w, b)


def kernel(x, adj_t, W1, b1, W2, b2):
    src = adj_t[0].astype(jnp.int32)
    dst = adj_t[1].astype(jnp.int32)
    pad = E_PAD - E
    pad_i = jnp.arange(pad, dtype=jnp.int32)
    src = jnp.concatenate([src, pad_i % N]).reshape(NW, NPH, CPP, K)
    dst = jnp.concatenate([dst, N + pad_i % (N_PAD - N)]).reshape(NW, NPH, CPP, K)
    zeros = jnp.zeros((ZR, D), jnp.float32)
    b1r = b1.reshape(1, D)
    b2r = b2.reshape(1, D)

    p1 = _sc_agg(x, src, dst, zeros)
    h = _linear(x, p1, W1, b1r, relu=True)
    p2 = _sc_agg(h, src, dst, zeros)
    return _linear(h, p2, W2, b2r, relu=False)


# speedup vs baseline: 1.3653x; 1.0152x over previous
import functools

import jax
import jax.numpy as jnp
from jax import lax
from jax.experimental import pallas as pl
from jax.experimental.pallas import tpu as pltpu
from jax.experimental.pallas import tpu_sc as plsc

N = 10000
E = 320000
D = 128
NC = 2
NS = 16
NW = NC * NS
K = 128
CHUNKS = 80
CPP = CHUNKS // 2
EPW = CHUNKS * K
E_PAD = EPW * NW
N_PAD = 10240
ZR = N_PAD // NS

_mesh = plsc.VectorSubcoreMesh(
    core_axis_name="c", subcore_axis_name="s", num_cores=NC, num_subcores=NS
)


@functools.partial(
    pl.kernel,
    out_type=jax.ShapeDtypeStruct((NC, N_PAD, D), jnp.float32),
    mesh=_mesh,
    scratch_types=[
        pltpu.VMEM((CPP, K), jnp.int32),
        pltpu.VMEM((CPP, K), jnp.int32),
        pltpu.VMEM((K, D), jnp.float32),
        pltpu.VMEM((K, D), jnp.float32),
        pltpu.VMEM_SHARED((N_PAD, D), jnp.float32),
        pltpu.SemaphoreType.DMA,
        pltpu.SemaphoreType.DMA,
    ],
)
def _sc_agg(h_hbm, src_hbm, dst_hbm, zeros_hbm, out_hbm,
            src_v, dst_v, rows0, rows1, acc, sem0, sem1):
    c = lax.axis_index("c")
    s = lax.axis_index("s")
    w = s * NC + c
    pltpu.sync_copy(zeros_hbm, acc.at[pl.ds(s * ZR, ZR)])

    def _fire(j, buf, sem):
        pltpu.async_copy(h_hbm.at[src_v.at[j]], buf, sem)

    def _drain_scatter(j, buf, sem):
        pltpu.make_async_copy(h_hbm.at[src_v.at[j]], buf, sem).wait()
        pltpu.sync_copy(buf, acc.at[dst_v.at[j]], add=True)

    plsc.subcore_barrier()

    for phase in range(CHUNKS // CPP):
        pltpu.sync_copy(src_hbm.at[w, pl.ds(phase * CPP, CPP)], src_v)
        pltpu.sync_copy(dst_hbm.at[w, pl.ds(phase * CPP, CPP)], dst_v)
        _fire(0, rows0, sem0)

        @pl.loop(0, CPP - 2, step=2)
        def _chunk(j0):
            _fire(j0 + 1, rows1, sem1)
            _drain_scatter(j0, rows0, sem0)
            _fire(j0 + 2, rows0, sem0)
            _drain_scatter(j0 + 1, rows1, sem1)

        _fire(CPP - 1, rows1, sem1)
        _drain_scatter(CPP - 2, rows0, sem0)
        _drain_scatter(CPP - 1, rows1, sem1)

    plsc.subcore_barrier()
    pltpu.sync_copy(acc.at[pl.ds(s * ZR, ZR)],
                    out_hbm.at[c, pl.ds(s * ZR, ZR)])


BN = 2000


def _lin_body(x_ref, p_ref, w_ref, b_ref, o_ref, *, relu):
    h = x_ref[...] + p_ref[0] + p_ref[1]
    y = lax.dot_general(h, w_ref[...], (((1,), (1,)), ((), ())),
                        preferred_element_type=jnp.float32)
    y = y + b_ref[...]
    if relu:
        y = jnp.maximum(y, 0.0)
    o_ref[...] = y


def _linear(x, p, w, b, relu):
    return pl.pallas_call(
        functools.partial(_lin_body, relu=relu),
        grid=(N // BN,),
        in_specs=[
            pl.BlockSpec((BN, D), lambda i: (i, 0)),
            pl.BlockSpec((NC, BN, D), lambda i: (0, i, 0)),
            pl.BlockSpec((D, D), lambda i: (0, 0)),
            pl.BlockSpec((1, D), lambda i: (0, 0)),
        ],
        out_specs=pl.BlockSpec((BN, D), lambda i: (i, 0)),
        out_shape=jax.ShapeDtypeStruct((N, D), jnp.float32),
    )(x, p, w, b)


def kernel(x, adj_t, W1, b1, W2, b2):
    src = adj_t[0].astype(jnp.int32)
    dst = adj_t[1].astype(jnp.int32)
    pad = E_PAD - E
    pad_i = jnp.arange(pad, dtype=jnp.int32)
    src = jnp.concatenate([src, pad_i % N]).reshape(NW, CHUNKS, K)
    dst = jnp.concatenate([dst, N + pad_i % (N_PAD - N)]).reshape(NW, CHUNKS, K)
    zeros = jnp.zeros((ZR, D), jnp.float32)
    b1r = b1.reshape(1, D)
    b2r = b2.reshape(1, D)

    p1 = _sc_agg(x, src, dst, zeros)
    h = _linear(x, p1, W1, b1r, relu=True)
    p2 = _sc_agg(h, src, dst, zeros)
    return _linear(h, p2, W2, b2r, relu=False)
